# speedup vs baseline: 1.2629x; 1.2629x over previous
import functools

import jax
import jax.numpy as jnp
from jax.experimental import pallas as pl
from jax.experimental.pallas import tpu as pltpu

CHUNK = 64


def _mm(a, b):
    return jax.lax.dot_general(a, b, (((1,), (0,)), ((), ())),
                               preferred_element_type=jnp.float32)


def _mmT(a, b):
    return jax.lax.dot_general(a, b, (((1,), (1,)), ((), ())),
                               preferred_element_type=jnp.float32)


def _mTm(a, b):
    return jax.lax.dot_general(a, b, (((0,), (0,)), ((), ())),
                               preferred_element_type=jnp.float32)


def _delta_kernel(x_ref, w_ref, b_ref, o_ref, scr_ref, *, L, D, hd):
    n_chunks = L // CHUNK
    rb = 512 if L % 512 == 0 else L
    w = w_ref[0]
    bias = b_ref[0]
    for r in range(L // rb):
        xs = x_ref[0, r * rb:(r + 1) * rb, :]
        scr_ref[r * rb:(r + 1) * rb, :] = _mmT(xs, w) + bias

    row = jax.lax.broadcasted_iota(jnp.int32, (CHUNK, CHUNK), 0)
    col = jax.lax.broadcasted_iota(jnp.int32, (CHUNK, CHUNK), 1)
    strict = row > col
    incl = row >= col
    eye = jnp.where(row == col, 1.0, 0.0).astype(jnp.float32)
    scale = hd ** -0.5

    def body(i, S):
        pc = scr_ref[pl.ds(i * CHUNK, CHUNK), :]
        q = pc[:, 0:hd] * scale
        k = pc[:, hd:2 * hd]
        v = pc[:, 2 * hd:3 * hd]
        beta = jax.nn.sigmoid(pc[:, 3 * hd:4 * hd])
        kb = k * beta
        vb = v * beta
        A = jnp.where(strict, _mmT(kb, k), 0.0)
        T = eye - A
        P = _mm(A, A)
        T = T + _mm(T, P)
        P = _mm(P, P)
        T = T + _mm(T, P)
        P = _mm(P, P)
        T = T + _mm(T, P)
        P = _mm(P, P)
        T = T + _mm(T, P)
        P = _mm(P, P)
        T = T + _mm(T, P)
        u = _mm(T, vb - _mm(kb, S))
        attn = jnp.where(incl, _mmT(q, k), 0.0)
        o = _mm(q, S) + _mm(attn, u)
        o_ref[0, pl.ds(i * CHUNK, CHUNK), 0, 0, :] = o
        return S + _mTm(k, u)

    jax.lax.fori_loop(0, n_chunks, body, jnp.zeros((hd, hd), jnp.float32))


def _out_kernel(o_ref, w_ref, b_ref, y_ref):
    y_ref[...] = _mmT(o_ref[...], w_ref[...]) + b_ref[...]


def kernel(hidden_states, Wq, bq, Wk, bk, Wv, bv, Wb, bb, Wo, bo):
    x = hidden_states
    B, L, D = x.shape
    H = Wb.shape[0]
    hd = D // H
    BH = B * H

    Wq_r = Wq.reshape(H, hd, D)
    Wk_r = Wk.reshape(H, hd, D)
    Wv_r = Wv.reshape(H, hd, D)
    Wb_r = jnp.broadcast_to(Wb[:, None, :], (H, hd, D))
    W_all = jnp.concatenate([Wq_r, Wk_r, Wv_r, Wb_r], axis=1)
    b_all = jnp.concatenate(
        [bq.reshape(H, hd), bk.reshape(H, hd), bv.reshape(H, hd),
         jnp.broadcast_to(bb[:, None], (H, hd))], axis=1).reshape(H, 1, 4 * hd)

    o_heads = pl.pallas_call(
        functools.partial(_delta_kernel, L=L, D=D, hd=hd),
        grid=(BH,),
        in_specs=[
            pl.BlockSpec((1, L, D), lambda i: (i // H, 0, 0)),
            pl.BlockSpec((1, 4 * hd, D), lambda i: (i % H, 0, 0)),
            pl.BlockSpec((1, 1, 4 * hd), lambda i: (i % H, 0, 0)),
        ],
        out_specs=pl.BlockSpec((1, L, 1, 1, hd), lambda i: (i // H, 0, i % H, 0, 0)),
        out_shape=jax.ShapeDtypeStruct((B, L, H, 1, hd), jnp.float32),
        scratch_shapes=[pltpu.VMEM((L, 4 * hd), jnp.float32)],
        compiler_params=pltpu.CompilerParams(
            dimension_semantics=("parallel",),
            vmem_limit_bytes=100 * 1024 * 1024,
        ),
    )(x, W_all, b_all)

    o_flat = o_heads.reshape(B * L, D)
    rb = 512 if (B * L) % 512 == 0 else B * L
    y = pl.pallas_call(
        _out_kernel,
        grid=((B * L) // rb,),
        in_specs=[
            pl.BlockSpec((rb, D), lambda i: (i, 0)),
            pl.BlockSpec((D, D), lambda i: (0, 0)),
            pl.BlockSpec((1, D), lambda i: (0, 0)),
        ],
        out_specs=pl.BlockSpec((rb, D), lambda i: (i, 0)),
        out_shape=jax.ShapeDtypeStruct((B * L, D), jnp.float32),
        compiler_params=pltpu.CompilerParams(
            dimension_semantics=("parallel",),
            vmem_limit_bytes=100 * 1024 * 1024,
        ),
    )(o_flat, Wo, bo.reshape(1, D))
    return y.reshape(B, L, D)
